# baseline (speedup 1.0000x reference)
import functools

import jax
import jax.numpy as jnp
from jax import lax
from jax.experimental import pallas as pl
from jax.experimental.pallas import tpu as pltpu
from jax.experimental.pallas import tpu_sc as plsc

R = 16384
J = 50
D = 64
NUM_WORKERS = 32
RB = 128
N_RB = R // RB
RB_PER_W = N_RB // NUM_WORKERS
V = 1000000


def _build_gather():
    mesh = plsc.VectorSubcoreMesh(core_axis_name="c", subcore_axis_name="s")

    @functools.partial(
        pl.kernel,
        mesh=mesh,
        out_type=jax.ShapeDtypeStruct((J, D, R), jnp.float32),
        scratch_types=[
            pltpu.VMEM((RB * J,), jnp.int32),
            pltpu.VMEM((J, RB), jnp.int32),
            pltpu.VMEM((J, RB), jnp.int32),
            pltpu.VMEM((4, RB, 128), jnp.float32),
            pltpu.VMEM((4, D, RB), jnp.float32),
            pltpu.SemaphoreType.DMA,
            pltpu.SemaphoreType.DMA,
        ],
        compiler_params=pltpu.CompilerParams(needs_layout_passes=False),
    )
    def gather_kernel(idx_hbm, table2_hbm, out_hbm,
                      slab_v, idxcol_v, off_v, rows_v, t_v, gsem, wsem):
        wid = lax.axis_index("s") * 2 + lax.axis_index("c")
        iota16 = lax.iota(jnp.int32, 16)

        def start_gather(j, b):
            pltpu.async_copy(table2_hbm.at[idxcol_v.at[j]], rows_v.at[b], gsem)

        def wait_gather(b):
            pltpu.make_async_copy(
                table2_hbm.at[idxcol_v.at[0]], rows_v.at[b], gsem
            ).wait()

        def start_write(j, rb, b):
            pltpu.async_copy(
                t_v.at[b], out_hbm.at[j, :, pl.ds(rb * RB, RB)], wsem
            )

        def wait_write(b):
            pltpu.make_async_copy(
                t_v.at[b], out_hbm.at[0, :, pl.ds(0, RB)], wsem
            ).wait()

        def transpose(j, b):
            for kg in range(8):
                kvec = kg * 16 + iota16
                offv = off_v[j, pl.ds(kg * 16, 16)]
                sl = pl.ds(kg * 16, 16)

                @plsc.parallel_loop(0, D, unroll=8)
                def _(d):
                    v = plsc.load_gather(rows_v.at[b], [kvec, offv + d])
                    t_v[b, d, sl] = v

        def per_rb(i, carry):
            rb = wid * RB_PER_W + i
            pltpu.sync_copy(idx_hbm.at[pl.ds(rb * RB * J, RB * J)], slab_v)

            def extract_j(j, c):
                for kg in range(8):
                    av = (kg * 16 + iota16) * J + j
                    v = plsc.load_gather(slab_v, [av])
                    idxcol_v[j, pl.ds(kg * 16, 16)] = v >> 1
                    off_v[j, pl.ds(kg * 16, 16)] = (v & 1) << 6
                return c
            lax.fori_loop(0, J, extract_j, 0)

            for j in range(3):
                start_gather(j, j)
            for j in range(4):
                wait_gather(j)
                start_gather(j + 3, (j + 3) % 4)
                transpose(j, j)
                start_write(j, rb, j)

            def body(i2, c):
                for u in range(4):
                    j = 4 * i2 + u
                    wait_gather(u)

                    @pl.when(j + 3 <= J - 1)
                    def _():
                        start_gather(j + 3, (u + 3) % 4)

                    wait_write(u)
                    transpose(j, u)
                    start_write(j, rb, u)
                return c
            lax.fori_loop(1, 12, body, 0)

            for j in (48, 49):
                b = j % 4
                wait_gather(b)
                wait_write(b)
                transpose(j, b)
                start_write(j, rb, b)
            for b in range(4):
                wait_write(b)
            return carry

        lax.fori_loop(0, RB_PER_W, per_rb, 0)

    return gather_kernel


def kernel(x, table):
    idx = x.reshape(R * J).astype(jnp.int32)
    table2 = table.reshape(V // 2, 128)
    out = _build_gather()(idx, table2)
    return out.transpose(2, 0, 1)
